# baseline (speedup 1.0000x reference)
import functools

import jax
import jax.numpy as jnp
from jax import lax
from jax.experimental import pallas as pl
from jax.experimental.pallas import tpu as pltpu
from jax.experimental.pallas import tpu_sc as plsc

D_EMBED = 64
D_MODEL = 1024
BATCH = 4
SEQ = 2048
B = BATCH * SEQ

NC, NS = 2, 16
NW = NC * NS
B_PER_W = B // NW
CHUNK = 128
NCHUNK = B_PER_W // CHUNK

_mesh = plsc.VectorSubcoreMesh(
    core_axis_name="c", subcore_axis_name="s", num_cores=NC, num_subcores=NS
)


@functools.partial(
    pl.kernel,
    out_type=jax.ShapeDtypeStruct((B // CHUNK, CHUNK, D_EMBED), jnp.float32),
    mesh=_mesh,
    scratch_types=[
        pltpu.VMEM((NCHUNK, CHUNK), jnp.int32),
        pltpu.VMEM((NCHUNK, CHUNK, D_EMBED), jnp.float32),
        pltpu.SemaphoreType.DMA,
    ],
    compiler_params=pltpu.CompilerParams(use_tc_tiling_on_sc=False),
)
def _sc_gather(table_hbm, idx_hbm, out_hbm, idx_v, rows_v, sem):
    wid = lax.axis_index("s") * NC + lax.axis_index("c")
    base = wid * NCHUNK
    pltpu.sync_copy(idx_hbm.at[pl.ds(base, NCHUNK)], idx_v)
    copies = [
        pltpu.async_copy(table_hbm.at[idx_v.at[j]], rows_v.at[j], sem)
        for j in range(NCHUNK)
    ]
    for c in copies:
        c.wait()
    pltpu.sync_copy(rows_v, out_hbm.at[pl.ds(base, NCHUNK)])


def _proj_body(x_ref, w_ref, o_ref):
    o_ref[...] = lax.dot_general(
        x_ref[...], w_ref[...],
        dimension_numbers=(((1,), (1,)), ((), ())),
        preferred_element_type=jnp.float32,
    )


ROW_BLK = 1024


def _project(x, w):
    return pl.pallas_call(
        _proj_body,
        grid=(B // ROW_BLK,),
        in_specs=[
            pl.BlockSpec((ROW_BLK, D_EMBED), lambda i: (i, 0)),
            pl.BlockSpec((D_MODEL, D_EMBED), lambda i: (0, 0)),
        ],
        out_specs=pl.BlockSpec((ROW_BLK, D_MODEL), lambda i: (i, 0)),
        out_shape=jax.ShapeDtypeStruct((B, D_MODEL), jnp.float32),
    )(x, w)


def kernel(input_ids, emb_table, emb_proj):
    idx = jnp.transpose(input_ids, (1, 0)).astype(jnp.int32)
    idx = idx.reshape(B // CHUNK, CHUNK)
    rows = _sc_gather(emb_table, idx)
    embed = _project(rows.reshape(B, D_EMBED), emb_proj)
    return embed.reshape(SEQ, BATCH, D_MODEL)


# speedup vs baseline: 1.0022x; 1.0022x over previous
import functools

import jax
import jax.numpy as jnp
from jax import lax
from jax.experimental import pallas as pl
from jax.experimental.pallas import tpu as pltpu
from jax.experimental.pallas import tpu_sc as plsc

D_EMBED = 64
D_MODEL = 1024
BATCH = 4
SEQ = 2048
B = BATCH * SEQ
PAIR_W = 2 * D_EMBED

NC, NS = 2, 16
NW = NC * NS
B_PER_W = B // NW
CHUNK = 128
NCHUNK = B_PER_W // CHUNK

_mesh = plsc.VectorSubcoreMesh(
    core_axis_name="c", subcore_axis_name="s", num_cores=NC, num_subcores=NS
)


@functools.partial(
    pl.kernel,
    out_type=jax.ShapeDtypeStruct((B // CHUNK, CHUNK, PAIR_W), jnp.float32),
    mesh=_mesh,
    scratch_types=[
        pltpu.VMEM((B_PER_W,), jnp.int32),
        pltpu.VMEM((NCHUNK, CHUNK, PAIR_W), jnp.float32),
        pltpu.SemaphoreType.DMA,
    ],
)
def _sc_gather(table_hbm, idx_hbm, out_hbm, idx_v, rows_v, sem):
    wid = lax.axis_index("s") * NC + lax.axis_index("c")
    pltpu.sync_copy(idx_hbm.at[pl.ds(wid * B_PER_W, B_PER_W)], idx_v)
    copies = [
        pltpu.async_copy(
            table_hbm.at[idx_v.at[pl.ds(j * CHUNK, CHUNK)]], rows_v.at[j], sem
        )
        for j in range(NCHUNK)
    ]
    for c in copies:
        c.wait()
    pltpu.sync_copy(rows_v, out_hbm.at[pl.ds(wid * NCHUNK, NCHUNK)])


ROW_BLK = 1024


def _proj_body(x_ref, p_ref, w2_ref, o_ref):
    x = x_ref[...]
    par = p_ref[...]
    lane = lax.broadcasted_iota(jnp.int32, (ROW_BLK, PAIR_W), 1)
    xm = jnp.where((lane // D_EMBED) == par, x, 0.0)
    o_ref[...] = lax.dot_general(
        xm, w2_ref[...],
        dimension_numbers=(((1,), (1,)), ((), ())),
        preferred_element_type=jnp.float32,
    )


def _project(x, parity, w2):
    return pl.pallas_call(
        _proj_body,
        grid=(B // ROW_BLK,),
        in_specs=[
            pl.BlockSpec((ROW_BLK, PAIR_W), lambda i: (i, 0)),
            pl.BlockSpec((ROW_BLK, 1), lambda i: (i, 0)),
            pl.BlockSpec((D_MODEL, PAIR_W), lambda i: (0, 0)),
        ],
        out_specs=pl.BlockSpec((ROW_BLK, D_MODEL), lambda i: (i, 0)),
        out_shape=jax.ShapeDtypeStruct((B, D_MODEL), jnp.float32),
    )(x, parity, w2)


def kernel(input_ids, emb_table, emb_proj):
    ids = jnp.transpose(input_ids, (1, 0)).astype(jnp.int32).reshape(B)
    pair_idx = ids >> 1
    parity = (ids & 1).reshape(B, 1)
    table2 = emb_table.reshape(emb_table.shape[0] // 2, PAIR_W)
    w2 = jnp.concatenate([emb_proj, emb_proj], axis=1)
    rows = _sc_gather(table2, pair_idx)
    embed = _project(rows.reshape(B, PAIR_W), parity, w2)
    return embed.reshape(SEQ, BATCH, D_MODEL)
